# baseline (speedup 1.0000x reference)
import functools

import jax
import jax.numpy as jnp
from jax import lax
from jax.experimental import pallas as pl
from jax.experimental.pallas import tpu as pltpu
from jax.experimental.pallas import tpu_sc as plsc

N_NODES = 10000
N_EDGES = 320000
D = 128
F = 64
K_HOPS = 5
NP = 10240
TPT = 640
NSUB = 5
EW = 128
CH = 160
G = 8
NG = CH // G
EPT = CH * EW
E_PAD = 16 * EPT

_f32 = jnp.float32
_i32 = jnp.int32

_mesh = plsc.VectorSubcoreMesh(
    core_axis_name="c", subcore_axis_name="s", num_cores=2, num_subcores=16)


def _deg_body(cols_t, dinv2, dis2, degsh, colv, onesb, degv, dinvv, disv):
    cid = lax.axis_index("c")
    sid = lax.axis_index("s")
    nbase = sid * TPT

    pltpu.sync_copy(cols_t.at[pl.ds(sid * CH, CH)], colv)

    def fill_ones_ew(r, _):
        onesb[r, :] = jnp.ones((16,), _f32)
        return 0
    lax.fori_loop(0, EW, fill_ones_ew, 0)

    def fill_ones640(r, _):
        degv[r, :] = jnp.ones((16,), _f32)
        return 0
    lax.fori_loop(0, TPT, fill_ones640, 0)
    pltpu.sync_copy(degv, degsh.at[pl.ds(nbase, TPT)])
    plsc.subcore_barrier()

    def scat(j, _):
        pltpu.sync_copy(onesb, degsh.at[colv.at[j]], add=True)
        return 0
    lax.fori_loop(0, CH, scat, 0)
    plsc.subcore_barrier()

    pltpu.sync_copy(degsh.at[pl.ds(nbase, TPT)], degv)

    def newton(r, _):
        d = degv[r, :]
        i = lax.bitcast_convert_type(d, _i32)
        i = jnp.int32(0x5F3759DF) - lax.shift_right_logical(i, 1)
        y = lax.bitcast_convert_type(i, _f32)
        for _ in range(4):
            y = y * (1.5 - 0.5 * d * y * y)
        disv[r, :] = y
        dinvv[r, :] = y * y
        return 0
    lax.fori_loop(0, TPT, newton, 0)

    pltpu.sync_copy(dinvv, dinv2.at[cid, pl.ds(nbase, TPT)])
    pltpu.sync_copy(disv, dis2.at[cid, pl.ds(nbase, TPT)])


_deg_kernel = pl.kernel(
    _deg_body,
    out_type=[jax.ShapeDtypeStruct((2, NP, 16), _f32),
              jax.ShapeDtypeStruct((2, NP, 16), _f32)],
    mesh=_mesh,
    scratch_types=[
        pltpu.VMEM_SHARED((NP, 16), _f32),
        pltpu.VMEM((CH, EW), _i32),
        pltpu.VMEM((EW, 16), _f32),
        pltpu.VMEM((TPT, 16), _f32),
        pltpu.VMEM((TPT, 16), _f32),
        pltpu.VMEM((TPT, 16), _f32),
    ],
    compiler_params=pltpu.CompilerParams(use_tc_tiling_on_sc=False),
)


def _diff_body(xh, rows_t, cols_t, dinv2, dis2, scv_h, outh,
               v_a, v_b, rowg, colg, dinb, disb,
               gbufa, gbufb, gbufc, gbufd, sem_i, sem_g, sem_s):
    vv, av = gbufa, gbufb
    cid = lax.axis_index("c")
    sid = lax.axis_index("s")
    nbase = sid * TPT

    pltpu.sync_copy(scv_h, disb.at[pl.ds(0, K_HOPS + 1)])
    svec = [disb[k, :] for k in range(K_HOPS + 1)]

    for sub in range(NSUB):
        base = nbase + sub * 128
        pltpu.sync_copy(xh.at[cid, pl.ds(base, 128)], vv.at[pl.ds(0, 128)])
        pltpu.sync_copy(dis2.at[cid, pl.ds(base, 128)], disb)

        def pro(r, _):
            dr = disb[r, :]
            for fb in range(4):
                fs = pl.ds(fb * 16, 16)
                vv[r, fs] = vv[r, fs] * dr
            return 0
        lax.fori_loop(0, 128, pro, 0)
        pltpu.sync_copy(vv.at[pl.ds(0, 128)], v_a.at[pl.ds(base, 128)])
        pltpu.sync_copy(vv.at[pl.ds(0, 128)], v_b.at[pl.ds(base, 128)])
    plsc.subcore_barrier()

    for k in range(1, K_HOPS + 1):
        src, dst = (v_a, v_b) if k % 2 == 1 else (v_b, v_a)

        ibase = sid * CH
        pltpu.async_copy(rows_t.at[pl.ds(ibase, G)], rowg.at[0], sem_i)
        pltpu.async_copy(cols_t.at[pl.ds(ibase, G)], colg.at[0], sem_i)

        def egroup(g, _, src=src, dst=dst):
            p = lax.rem(g, 2)
            pltpu.make_async_copy(
                rows_t.at[pl.ds(ibase, G)], rowg.at[p], sem_i).wait()
            pltpu.make_async_copy(
                cols_t.at[pl.ds(ibase, G)], colg.at[p], sem_i).wait()

            @pl.when(g < NG - 1)
            def _():
                off = ibase + (g + 1) * G
                pltpu.async_copy(rows_t.at[pl.ds(off, G)],
                                 rowg.at[1 - p], sem_i)
                pltpu.async_copy(cols_t.at[pl.ds(off, G)],
                                 colg.at[1 - p], sem_i)

            gb = (gbufa, gbufb, gbufc, gbufd)
            gd = [None] * G
            sd = [None] * G
            gd[0] = pltpu.async_copy(src.at[rowg.at[p, 0]], gb[0], sem_g)
            gd[1] = pltpu.async_copy(src.at[rowg.at[p, 1]], gb[1], sem_g)
            for c in range(G):
                gd[c].wait()
                sd[c] = pltpu.async_copy(
                    gb[c % 4], dst.at[colg.at[p, c]], sem_s, add=True)
                if c + 2 < G:
                    if c >= 2:
                        sd[c - 2].wait()
                    gd[c + 2] = pltpu.async_copy(
                        src.at[rowg.at[p, c + 2]], gb[(c + 2) % 4], sem_g)
            for c in range(max(0, G - 4), G):
                sd[c].wait()
            return 0
        lax.fori_loop(0, NG, egroup, 0)
        plsc.subcore_barrier()

        last = (k == K_HOPS)
        for sub in range(NSUB):
            base = nbase + sub * 128
            pltpu.sync_copy(dst.at[pl.ds(base, 128)], vv.at[pl.ds(0, 128)])
            if not last:
                pltpu.sync_copy(dinv2.at[cid, pl.ds(base, 128)], dinb)
            if k > 1:
                pltpu.sync_copy(outh.at[cid, pl.ds(base, 128)], av.at[pl.ds(0, 128)])

            def node(r, _, k=k, last=last):
                dr = dinb[r, :]
                s = svec[k]
                for fb in range(4):
                    fs = pl.ds(fb * 16, 16)
                    v16 = vv[r, fs]
                    if k == 1:
                        av[r, fs] = s * v16
                    else:
                        av[r, fs] = av[r, fs] + s * v16
                    if not last:
                        vv[r, fs] = dr * v16
                return 0
            lax.fori_loop(0, 128, node, 0)
            pltpu.sync_copy(av.at[pl.ds(0, 128)], outh.at[cid, pl.ds(base, 128)])
            if not last:
                pltpu.sync_copy(vv.at[pl.ds(0, 128)], dst.at[pl.ds(base, 128)])
                pltpu.sync_copy(vv.at[pl.ds(0, 128)], src.at[pl.ds(base, 128)])
        plsc.subcore_barrier()

    for sub in range(NSUB):
        base = nbase + sub * 128
        pltpu.sync_copy(xh.at[cid, pl.ds(base, 128)], vv.at[pl.ds(0, 128)])
        pltpu.sync_copy(dis2.at[cid, pl.ds(base, 128)], disb)
        pltpu.sync_copy(outh.at[cid, pl.ds(base, 128)], av.at[pl.ds(0, 128)])

        def epi(r, _):
            dr = disb[r, :]
            s0 = svec[0]
            for fb in range(4):
                fs = pl.ds(fb * 16, 16)
                av[r, fs] = s0 * vv[r, fs] + dr * av[r, fs]
            return 0
        lax.fori_loop(0, 128, epi, 0)
        pltpu.sync_copy(av.at[pl.ds(0, 128)], outh.at[cid, pl.ds(base, 128)])


_diff_kernel = pl.kernel(
    _diff_body,
    out_type=jax.ShapeDtypeStruct((2, NP, F), _f32),
    mesh=_mesh,
    scratch_types=[
        pltpu.VMEM_SHARED((NP, F), _f32),
        pltpu.VMEM_SHARED((NP, F), _f32),
        pltpu.VMEM((2, G, EW), _i32),
        pltpu.VMEM((2, G, EW), _i32),
        pltpu.VMEM((128, 16), _f32),
        pltpu.VMEM((128, 16), _f32),
        pltpu.VMEM((EW, F), _f32),
        pltpu.VMEM((EW, F), _f32),
        pltpu.VMEM((EW, F), _f32),
        pltpu.VMEM((EW, F), _f32),
        pltpu.SemaphoreType.DMA,
        pltpu.SemaphoreType.DMA,
        pltpu.SemaphoreType.DMA,
    ],
    compiler_params=pltpu.CompilerParams(use_tc_tiling_on_sc=False),
)


def _mm_body(d_ref, w_ref, b_ref, o_ref, *, elu):
    y = lax.dot_general(d_ref[...], w_ref[...], (((1,), (1,)), ((), ())),
                        preferred_element_type=_f32)
    y = y + b_ref[...]
    if elu:
        y = jnp.where(y > 0, y, jnp.exp(y) - 1.0)
    o_ref[...] = y


def _mm(d, w, b, elu):
    return pl.pallas_call(
        functools.partial(_mm_body, elu=elu),
        grid=(NP // 512,),
        in_specs=[
            pl.BlockSpec((512, D), lambda i: (i, 0)),
            pl.BlockSpec((D, D), lambda i: (0, 0)),
            pl.BlockSpec((1, D), lambda i: (0, 0)),
        ],
        out_specs=pl.BlockSpec((512, D), lambda i: (i, 0)),
        out_shape=jax.ShapeDtypeStruct((NP, D), _f32),
    )(d, w, b.reshape(1, D))


def _scales(log_t):
    t = jnp.exp(log_t)
    sc = [jnp.exp(-t)]
    for k in range(1, K_HOPS + 1):
        sc.append(sc[-1] * t / k)
    return jnp.broadcast_to(jnp.stack(sc)[:, None], (K_HOPS + 1, 16))


def _split(h):
    return h.reshape(NP, 2, F).transpose(1, 0, 2)


def _join(hh):
    return hh.transpose(1, 0, 2).reshape(NP, D)


@jax.jit
def kernel(x, edge_index, log_t1, W1, b1, log_t2, W2, b2):
    ei = jnp.asarray(edge_index, _i32)
    pad = E_PAD - N_EDGES
    fill = jnp.full((pad,), NP - 1, _i32)
    rows_t = jnp.concatenate([ei[0], fill]).reshape(16 * CH, EW)
    cols_t = jnp.concatenate([ei[1], fill]).reshape(16 * CH, EW)

    xp = jnp.pad(x, ((0, NP - N_NODES), (0, 0)))

    dinv2, dis2 = _deg_kernel(cols_t)

    d1 = _join(_diff_kernel(_split(xp), rows_t, cols_t, dinv2, dis2,
                            _scales(log_t1)))
    z = _mm(d1, W1, b1, elu=True)
    d2 = _join(_diff_kernel(_split(z), rows_t, cols_t, dinv2, dis2,
                            _scales(log_t2)))
    out = _mm(d2, W2, b2, elu=False)
    return out[:N_NODES]


# speedup vs baseline: 1.0452x; 1.0452x over previous
import functools

import jax
import jax.numpy as jnp
from jax import lax
from jax.experimental import pallas as pl
from jax.experimental.pallas import tpu as pltpu
from jax.experimental.pallas import tpu_sc as plsc

N_NODES = 10000
N_EDGES = 320000
D = 128
F = 64
K_HOPS = 5
NP = 10240
TPT = 640
NSUB = 5
EW = 192
CH = 105
G = 7
NG = CH // G
EPT = CH * EW
E_PAD = 16 * EPT

_f32 = jnp.float32
_i32 = jnp.int32

_mesh = plsc.VectorSubcoreMesh(
    core_axis_name="c", subcore_axis_name="s", num_cores=2, num_subcores=16)


def _deg_body(cols_t, dinv2, dis2, degsh, colv, onesb, degv, dinvv, disv):
    cid = lax.axis_index("c")
    sid = lax.axis_index("s")
    nbase = sid * TPT

    pltpu.sync_copy(cols_t.at[pl.ds(sid * CH, CH)], colv)

    def fill_ones_ew(r, _):
        onesb[r, :] = jnp.ones((16,), _f32)
        return 0
    lax.fori_loop(0, EW, fill_ones_ew, 0)

    def fill_ones640(r, _):
        degv[r, :] = jnp.ones((16,), _f32)
        return 0
    lax.fori_loop(0, TPT, fill_ones640, 0)
    pltpu.sync_copy(degv, degsh.at[pl.ds(nbase, TPT)])
    plsc.subcore_barrier()

    def scat(j, _):
        pltpu.sync_copy(onesb, degsh.at[colv.at[j]], add=True)
        return 0
    lax.fori_loop(0, CH, scat, 0)
    plsc.subcore_barrier()

    pltpu.sync_copy(degsh.at[pl.ds(nbase, TPT)], degv)

    def newton(r, _):
        d = degv[r, :]
        i = lax.bitcast_convert_type(d, _i32)
        i = jnp.int32(0x5F3759DF) - lax.shift_right_logical(i, 1)
        y = lax.bitcast_convert_type(i, _f32)
        for _ in range(4):
            y = y * (1.5 - 0.5 * d * y * y)
        disv[r, :] = y
        dinvv[r, :] = y * y
        return 0
    lax.fori_loop(0, TPT, newton, 0)

    pltpu.sync_copy(dinvv, dinv2.at[cid, pl.ds(nbase, TPT)])
    pltpu.sync_copy(disv, dis2.at[cid, pl.ds(nbase, TPT)])


_deg_kernel = pl.kernel(
    _deg_body,
    out_type=[jax.ShapeDtypeStruct((2, NP, 16), _f32),
              jax.ShapeDtypeStruct((2, NP, 16), _f32)],
    mesh=_mesh,
    scratch_types=[
        pltpu.VMEM_SHARED((NP, 16), _f32),
        pltpu.VMEM((CH, EW), _i32),
        pltpu.VMEM((EW, 16), _f32),
        pltpu.VMEM((TPT, 16), _f32),
        pltpu.VMEM((TPT, 16), _f32),
        pltpu.VMEM((TPT, 16), _f32),
    ],
    compiler_params=pltpu.CompilerParams(use_tc_tiling_on_sc=False),
)


def _diff_body(xsh, rows_t, cols_t, dinv2, scv_h, outh,
               v_a, v_b, rowg, colg, dinb,
               gbufa, gbufb, gbufc, sem_i, sem_g, sem_s):
    vv, av = gbufa, gbufb
    cid = lax.axis_index("c")
    sid = lax.axis_index("s")
    nbase = sid * TPT

    pltpu.sync_copy(scv_h, dinb.at[pl.ds(0, K_HOPS + 1)])
    svec = [dinb[k, :] for k in range(K_HOPS + 1)]

    pltpu.sync_copy(xsh.at[cid, pl.ds(nbase, TPT)], v_a.at[pl.ds(nbase, TPT)])
    pltpu.sync_copy(xsh.at[cid, pl.ds(nbase, TPT)], v_b.at[pl.ds(nbase, TPT)])
    plsc.subcore_barrier()

    for k in range(1, K_HOPS + 1):
        src, dst = (v_a, v_b) if k % 2 == 1 else (v_b, v_a)

        ibase = sid * CH
        pltpu.async_copy(rows_t.at[pl.ds(ibase, G)], rowg.at[0], sem_i)
        pltpu.async_copy(cols_t.at[pl.ds(ibase, G)], colg.at[0], sem_i)

        def egroup(g, _, src=src, dst=dst):
            p = lax.rem(g, 2)
            pltpu.make_async_copy(
                rows_t.at[pl.ds(ibase, G)], rowg.at[p], sem_i).wait()
            pltpu.make_async_copy(
                cols_t.at[pl.ds(ibase, G)], colg.at[p], sem_i).wait()

            @pl.when(g < NG - 1)
            def _():
                off = ibase + (g + 1) * G
                pltpu.async_copy(rows_t.at[pl.ds(off, G)],
                                 rowg.at[1 - p], sem_i)
                pltpu.async_copy(cols_t.at[pl.ds(off, G)],
                                 colg.at[1 - p], sem_i)

            gb = (gbufa, gbufb, gbufc)
            gd = [None] * G
            sd = [None] * G
            gd[0] = pltpu.async_copy(src.at[rowg.at[p, 0]], gb[0], sem_g)
            gd[1] = pltpu.async_copy(src.at[rowg.at[p, 1]], gb[1], sem_g)
            for c in range(G):
                gd[c].wait()
                sd[c] = pltpu.async_copy(
                    gb[c % 3], dst.at[colg.at[p, c]], sem_s, add=True)
                if c + 2 < G:
                    if c >= 1:
                        sd[c - 1].wait()
                    gd[c + 2] = pltpu.async_copy(
                        src.at[rowg.at[p, c + 2]], gb[(c + 2) % 3], sem_g)
            for c in range(max(0, G - 3), G):
                sd[c].wait()
            return 0
        lax.fori_loop(0, NG, egroup, 0)
        plsc.subcore_barrier()

        last = (k == K_HOPS)
        for sub in range(NSUB):
            base = nbase + sub * 128
            pltpu.sync_copy(dst.at[pl.ds(base, 128)], vv.at[pl.ds(0, 128)])
            if not last:
                pltpu.sync_copy(dinv2.at[cid, pl.ds(base, 128)], dinb)
            if k > 1:
                pltpu.sync_copy(outh.at[cid, pl.ds(base, 128)], av.at[pl.ds(0, 128)])

            def node(r, _, k=k, last=last):
                dr = dinb[r, :]
                s = svec[k]
                for fb in range(4):
                    fs = pl.ds(fb * 16, 16)
                    v16 = vv[r, fs]
                    if k == 1:
                        av[r, fs] = s * v16
                    else:
                        av[r, fs] = av[r, fs] + s * v16
                    if not last:
                        vv[r, fs] = dr * v16
                return 0
            lax.fori_loop(0, 128, node, 0)
            pltpu.sync_copy(av.at[pl.ds(0, 128)], outh.at[cid, pl.ds(base, 128)])
            if not last:
                pltpu.sync_copy(vv.at[pl.ds(0, 128)], dst.at[pl.ds(base, 128)])
                pltpu.sync_copy(vv.at[pl.ds(0, 128)], src.at[pl.ds(base, 128)])
        plsc.subcore_barrier()


_diff_kernel = pl.kernel(
    _diff_body,
    out_type=jax.ShapeDtypeStruct((2, NP, F), _f32),
    mesh=_mesh,
    scratch_types=[
        pltpu.VMEM_SHARED((NP, F), _f32),
        pltpu.VMEM_SHARED((NP, F), _f32),
        pltpu.VMEM((2, G, EW), _i32),
        pltpu.VMEM((2, G, EW), _i32),
        pltpu.VMEM((128, 16), _f32),
        pltpu.VMEM((EW, F), _f32),
        pltpu.VMEM((EW, F), _f32),
        pltpu.VMEM((EW, F), _f32),
        pltpu.SemaphoreType.DMA,
        pltpu.SemaphoreType.DMA,
        pltpu.SemaphoreType.DMA,
    ],
    compiler_params=pltpu.CompilerParams(use_tc_tiling_on_sc=False),
)


def _mm_body(acc_ref, x_ref, dis_ref, s0_ref, w_ref, b_ref, o_ref, os_ref,
             *, elu, also_scale):
    d = s0_ref[0, 0] * x_ref[...] + dis_ref[...] * acc_ref[...]
    y = lax.dot_general(d, w_ref[...], (((1,), (1,)), ((), ())),
                        preferred_element_type=_f32)
    y = y + b_ref[...]
    if elu:
        y = jnp.where(y > 0, y, jnp.exp(y) - 1.0)
    o_ref[...] = y
    if also_scale:
        os_ref[...] = dis_ref[...] * y


def _mm(acc, xin, dis, s0, w, b, elu, also_scale):
    blk = 512
    outs = [jax.ShapeDtypeStruct((NP, D), _f32),
            jax.ShapeDtypeStruct((NP, D), _f32)]
    o, os_ = pl.pallas_call(
        functools.partial(_mm_body, elu=elu, also_scale=also_scale),
        grid=(NP // blk,),
        in_specs=[
            pl.BlockSpec((blk, D), lambda i: (i, 0)),
            pl.BlockSpec((blk, D), lambda i: (i, 0)),
            pl.BlockSpec((blk, 1), lambda i: (i, 0)),
            pl.BlockSpec((1, 1), lambda i: (0, 0)),
            pl.BlockSpec((D, D), lambda i: (0, 0)),
            pl.BlockSpec((1, D), lambda i: (0, 0)),
        ],
        out_specs=[pl.BlockSpec((blk, D), lambda i: (i, 0)),
                   pl.BlockSpec((blk, D), lambda i: (i, 0))],
        out_shape=outs,
    )(acc, xin, dis, s0.reshape(1, 1), w, b.reshape(1, D))
    return o, os_


def _prescale_body(x_ref, dis_ref, o_ref):
    o_ref[...] = dis_ref[...] * x_ref[...]


def _prescale(x, dis):
    blk = 512
    return pl.pallas_call(
        _prescale_body,
        grid=(NP // blk,),
        in_specs=[pl.BlockSpec((blk, D), lambda i: (i, 0)),
                  pl.BlockSpec((blk, 1), lambda i: (i, 0))],
        out_specs=pl.BlockSpec((blk, D), lambda i: (i, 0)),
        out_shape=jax.ShapeDtypeStruct((NP, D), _f32),
    )(x, dis)


def _scales(log_t):
    t = jnp.exp(log_t)
    sc = [jnp.exp(-t)]
    for k in range(1, K_HOPS + 1):
        sc.append(sc[-1] * t / k)
    return jnp.stack(sc)


def _split(h):
    return h.reshape(NP, 2, F).transpose(1, 0, 2)


def _join(hh):
    return hh.transpose(1, 0, 2).reshape(NP, D)


@jax.jit
def kernel(x, edge_index, log_t1, W1, b1, log_t2, W2, b2):
    ei = jnp.asarray(edge_index, _i32)
    pad = E_PAD - N_EDGES
    fill = jnp.full((pad,), NP - 1, _i32)
    rows_t = jnp.concatenate([ei[0], fill]).reshape(16 * CH, EW)
    cols_t = jnp.concatenate([ei[1], fill]).reshape(16 * CH, EW)

    xp = jnp.pad(x, ((0, NP - N_NODES), (0, 0)))

    dinv2, dis2 = _deg_kernel(cols_t)
    dis_col = dis2[0, :, 0:1]

    s1 = _scales(log_t1)
    s2 = _scales(log_t2)
    sv1 = jnp.broadcast_to(s1[:, None], (K_HOPS + 1, 16))
    sv2 = jnp.broadcast_to(s2[:, None], (K_HOPS + 1, 16))

    xs1 = _prescale(xp, dis_col)
    acc1 = _join(_diff_kernel(_split(xs1), rows_t, cols_t, dinv2, sv1))
    h1, xs2 = _mm(acc1, xp, dis_col, s1[0], W1, b1,
                  elu=True, also_scale=True)
    acc2 = _join(_diff_kernel(_split(xs2), rows_t, cols_t, dinv2, sv2))
    out, _ = _mm(acc2, h1, dis_col, s2[0], W2, b2,
                 elu=False, also_scale=False)
    return out[:N_NODES]


# speedup vs baseline: 1.0651x; 1.0190x over previous
import functools

import jax
import jax.numpy as jnp
from jax import lax
from jax.experimental import pallas as pl
from jax.experimental.pallas import tpu as pltpu
from jax.experimental.pallas import tpu_sc as plsc

N_NODES = 10000
N_EDGES = 320000
D = 128
F = 64
K_HOPS = 5
NP = 10240
TPT = 640
NSUB = 5
EW = 192
CH = 105
G = 7
NG = CH // G
EPT = CH * EW
E_PAD = 16 * EPT

_f32 = jnp.float32
_i32 = jnp.int32

_mesh = plsc.VectorSubcoreMesh(
    core_axis_name="c", subcore_axis_name="s", num_cores=2, num_subcores=16)


def _deg_body(cols_t, dinv2, dis2, degsh, colv, onesb, degv, dinvv, disv):
    cid = lax.axis_index("c")
    sid = lax.axis_index("s")
    nbase = sid * TPT

    pltpu.sync_copy(cols_t.at[pl.ds(sid * CH, CH)], colv)

    def fill_ones_ew(r, _):
        onesb[r, :] = jnp.ones((16,), _f32)
        return 0
    lax.fori_loop(0, EW, fill_ones_ew, 0)

    def fill_ones640(r, _):
        degv[r, :] = jnp.ones((16,), _f32)
        return 0
    lax.fori_loop(0, TPT, fill_ones640, 0)
    pltpu.sync_copy(degv, degsh.at[pl.ds(nbase, TPT)])
    plsc.subcore_barrier()

    def scat(j, _):
        pltpu.sync_copy(onesb, degsh.at[colv.at[j]], add=True)
        return 0
    lax.fori_loop(0, CH, scat, 0)
    plsc.subcore_barrier()

    pltpu.sync_copy(degsh.at[pl.ds(nbase, TPT)], degv)

    def newton(r, _):
        d = degv[r, :]
        i = lax.bitcast_convert_type(d, _i32)
        i = jnp.int32(0x5F3759DF) - lax.shift_right_logical(i, 1)
        y = lax.bitcast_convert_type(i, _f32)
        for _ in range(4):
            y = y * (1.5 - 0.5 * d * y * y)
        disv[r, :] = y
        dinvv[r, :] = y * y
        return 0
    lax.fori_loop(0, TPT, newton, 0)

    pltpu.sync_copy(dinvv, dinv2.at[cid, pl.ds(nbase, TPT)])
    pltpu.sync_copy(disv, dis2.at[cid, pl.ds(nbase, TPT)])


_deg_kernel = pl.kernel(
    _deg_body,
    out_type=[jax.ShapeDtypeStruct((2, NP, 16), _f32),
              jax.ShapeDtypeStruct((2, NP, 16), _f32)],
    mesh=_mesh,
    scratch_types=[
        pltpu.VMEM_SHARED((NP, 16), _f32),
        pltpu.VMEM((CH, EW), _i32),
        pltpu.VMEM((EW, 16), _f32),
        pltpu.VMEM((TPT, 16), _f32),
        pltpu.VMEM((TPT, 16), _f32),
        pltpu.VMEM((TPT, 16), _f32),
    ],
    compiler_params=pltpu.CompilerParams(use_tc_tiling_on_sc=False),
)


def _diff_body(xsh, rows_t, cols_t, dinv2, scv_h, outh,
               v_a, v_b, rowg, colg, dinb,
               gbufa, gbufb, gbufc, sem_i, sem_g, sem_s):
    vv, av = gbufa, gbufb
    cid = lax.axis_index("c")
    sid = lax.axis_index("s")
    nbase = sid * TPT

    pltpu.sync_copy(scv_h, dinb.at[pl.ds(0, K_HOPS + 1)])
    svec = [dinb[k, :] for k in range(K_HOPS + 1)]

    pltpu.sync_copy(xsh.at[cid, pl.ds(nbase, TPT)], v_a.at[pl.ds(nbase, TPT)])
    pltpu.sync_copy(xsh.at[cid, pl.ds(nbase, TPT)], v_b.at[pl.ds(nbase, TPT)])
    plsc.subcore_barrier()

    for k in range(1, K_HOPS + 1):
        src, dst = (v_a, v_b) if k % 2 == 1 else (v_b, v_a)

        ibase = sid * CH
        pltpu.async_copy(rows_t.at[pl.ds(ibase, G)], rowg.at[0], sem_i)
        pltpu.async_copy(cols_t.at[pl.ds(ibase, G)], colg.at[0], sem_i)

        def egroup(g, _, src=src, dst=dst):
            p = lax.rem(g, 2)
            pltpu.make_async_copy(
                rows_t.at[pl.ds(ibase, G)], rowg.at[p], sem_i).wait()
            pltpu.make_async_copy(
                cols_t.at[pl.ds(ibase, G)], colg.at[p], sem_i).wait()

            @pl.when(g < NG - 1)
            def _():
                off = ibase + (g + 1) * G
                pltpu.async_copy(rows_t.at[pl.ds(off, G)],
                                 rowg.at[1 - p], sem_i)
                pltpu.async_copy(cols_t.at[pl.ds(off, G)],
                                 colg.at[1 - p], sem_i)

            gb = (gbufa, gbufb, gbufc)
            gd = [None] * G
            sd = [None] * G
            gd[0] = pltpu.async_copy(src.at[rowg.at[p, 0]], gb[0], sem_g)
            gd[1] = pltpu.async_copy(src.at[rowg.at[p, 1]], gb[1], sem_g)
            for c in range(G):
                gd[c].wait()
                sd[c] = pltpu.async_copy(
                    gb[c % 3], dst.at[colg.at[p, c]], sem_s, add=True)
                if c + 2 < G:
                    if c >= 1:
                        sd[c - 1].wait()
                    gd[c + 2] = pltpu.async_copy(
                        src.at[rowg.at[p, c + 2]], gb[(c + 2) % 3], sem_g)
            for c in range(max(0, G - 3), G):
                sd[c].wait()
            return 0
        lax.fori_loop(0, NG, egroup, 0)
        plsc.subcore_barrier()

        last = (k == K_HOPS)
        for sub in range(NSUB):
            base = nbase + sub * 128
            pltpu.sync_copy(dst.at[pl.ds(base, 128)], vv.at[pl.ds(0, 128)])
            if not last:
                pltpu.sync_copy(dinv2.at[cid, pl.ds(base, 128)], dinb)
            if k > 1:
                pltpu.sync_copy(outh.at[cid, pl.ds(base, 128)], av.at[pl.ds(0, 128)])

            def node(r, _, k=k, last=last):
                dr = dinb[r, :]
                s = svec[k]
                for fb in range(4):
                    fs = pl.ds(fb * 16, 16)
                    v16 = vv[r, fs]
                    if k == 1:
                        av[r, fs] = s * v16
                    else:
                        av[r, fs] = av[r, fs] + s * v16
                    if not last:
                        vv[r, fs] = dr * v16
                return 0
            lax.fori_loop(0, 128, node, 0)
            pltpu.sync_copy(av.at[pl.ds(0, 128)], outh.at[cid, pl.ds(base, 128)])
            if not last:
                pltpu.sync_copy(vv.at[pl.ds(0, 128)], dst.at[pl.ds(base, 128)])
                pltpu.sync_copy(vv.at[pl.ds(0, 128)], src.at[pl.ds(base, 128)])
        plsc.subcore_barrier()


_diff_kernel = pl.kernel(
    _diff_body,
    out_type=jax.ShapeDtypeStruct((2, NP, F), _f32),
    mesh=_mesh,
    scratch_types=[
        pltpu.VMEM_SHARED((NP, F), _f32),
        pltpu.VMEM_SHARED((NP, F), _f32),
        pltpu.VMEM((2, G, EW), _i32),
        pltpu.VMEM((2, G, EW), _i32),
        pltpu.VMEM((128, 16), _f32),
        pltpu.VMEM((EW, F), _f32),
        pltpu.VMEM((EW, F), _f32),
        pltpu.VMEM((EW, F), _f32),
        pltpu.SemaphoreType.DMA,
        pltpu.SemaphoreType.DMA,
        pltpu.SemaphoreType.DMA,
    ],
    compiler_params=pltpu.CompilerParams(use_tc_tiling_on_sc=False),
)


def _mm_body(acc_ref, x_ref, dis_ref, s0_ref, w_ref, b_ref, o_ref, os_ref,
             *, elu, also_scale):
    acc = jnp.concatenate([acc_ref[0], acc_ref[1]], axis=1)
    d = s0_ref[0, 0] * x_ref[...] + dis_ref[...] * acc
    y = lax.dot_general(d, w_ref[...], (((1,), (1,)), ((), ())),
                        preferred_element_type=_f32)
    y = y + b_ref[...]
    if elu:
        y = jnp.where(y > 0, y, jnp.exp(y) - 1.0)
    o_ref[...] = y
    if also_scale:
        ys = dis_ref[...] * y
        os_ref[0] = ys[:, :F]
        os_ref[1] = ys[:, F:]


def _mm(acch, xin, dis, s0, w, b, elu, also_scale):
    blk = 512
    outs = [jax.ShapeDtypeStruct((NP, D), _f32),
            jax.ShapeDtypeStruct((2, NP, F), _f32)]
    o, osh = pl.pallas_call(
        functools.partial(_mm_body, elu=elu, also_scale=also_scale),
        grid=(NP // blk,),
        in_specs=[
            pl.BlockSpec((2, blk, F), lambda i: (0, i, 0)),
            pl.BlockSpec((blk, D), lambda i: (i, 0)),
            pl.BlockSpec((blk, 1), lambda i: (i, 0)),
            pl.BlockSpec((1, 1), lambda i: (0, 0)),
            pl.BlockSpec((D, D), lambda i: (0, 0)),
            pl.BlockSpec((1, D), lambda i: (0, 0)),
        ],
        out_specs=[pl.BlockSpec((blk, D), lambda i: (i, 0)),
                   pl.BlockSpec((2, blk, F), lambda i: (0, i, 0))],
        out_shape=outs,
    )(acch, xin, dis, s0.reshape(1, 1), w, b.reshape(1, D))
    return o, osh


def _prescale_body(x_ref, dis_ref, o_ref):
    ys = dis_ref[...] * x_ref[...]
    o_ref[0] = ys[:, :F]
    o_ref[1] = ys[:, F:]


def _prescale(x, dis):
    blk = 512
    return pl.pallas_call(
        _prescale_body,
        grid=(NP // blk,),
        in_specs=[pl.BlockSpec((blk, D), lambda i: (i, 0)),
                  pl.BlockSpec((blk, 1), lambda i: (i, 0))],
        out_specs=pl.BlockSpec((2, blk, F), lambda i: (0, i, 0)),
        out_shape=jax.ShapeDtypeStruct((2, NP, F), _f32),
    )(x, dis)


def _scales(log_t):
    t = jnp.exp(log_t)
    sc = [jnp.exp(-t)]
    for k in range(1, K_HOPS + 1):
        sc.append(sc[-1] * t / k)
    return jnp.stack(sc)


@jax.jit
def kernel(x, edge_index, log_t1, W1, b1, log_t2, W2, b2):
    ei = jnp.asarray(edge_index, _i32)
    pad = E_PAD - N_EDGES
    fill = jnp.full((pad,), NP - 1, _i32)
    rows_t = jnp.concatenate([ei[0], fill]).reshape(16 * CH, EW)
    cols_t = jnp.concatenate([ei[1], fill]).reshape(16 * CH, EW)

    xp = jnp.pad(x, ((0, NP - N_NODES), (0, 0)))

    dinv2, dis2 = _deg_kernel(cols_t)
    dis_col = dis2[0, :, 0:1]

    s1 = _scales(log_t1)
    s2 = _scales(log_t2)
    sv1 = jnp.broadcast_to(s1[:, None], (K_HOPS + 1, 16))
    sv2 = jnp.broadcast_to(s2[:, None], (K_HOPS + 1, 16))

    xsh1 = _prescale(xp, dis_col)
    acch1 = _diff_kernel(xsh1, rows_t, cols_t, dinv2, sv1)
    h1, xsh2 = _mm(acch1, xp, dis_col, s1[0], W1, b1,
                   elu=True, also_scale=True)
    acch2 = _diff_kernel(xsh2, rows_t, cols_t, dinv2, sv2)
    out, _ = _mm(acch2, h1, dis_col, s2[0], W2, b2,
                 elu=False, also_scale=False)
    return out[:N_NODES]


# speedup vs baseline: 1.1049x; 1.0374x over previous
import functools

import jax
import jax.numpy as jnp
from jax import lax
from jax.experimental import pallas as pl
from jax.experimental.pallas import tpu as pltpu
from jax.experimental.pallas import tpu_sc as plsc

N_NODES = 10000
N_EDGES = 320000
D = 128
F = 64
K_HOPS = 5
NP = 10240
TPT = 640
NSUB = 5
EW = 192
CH = 105
G = 7
NG = CH // G
EPT = CH * EW
E_PAD = 16 * EPT

_f32 = jnp.float32
_i32 = jnp.int32

_mesh = plsc.VectorSubcoreMesh(
    core_axis_name="c", subcore_axis_name="s", num_cores=2, num_subcores=16)


def _deg_body(cols_t, dinv2, dis2, degsh, colv, onesb, degv, dinvv, disv):
    cid = lax.axis_index("c")
    sid = lax.axis_index("s")
    nbase = sid * TPT

    pltpu.sync_copy(cols_t.at[pl.ds(sid * CH, CH)], colv)

    def fill_ones_ew(r, _):
        onesb[r, :] = jnp.ones((16,), _f32)
        return 0
    lax.fori_loop(0, EW, fill_ones_ew, 0)

    def fill_ones640(r, _):
        degv[r, :] = jnp.ones((16,), _f32)
        return 0
    lax.fori_loop(0, TPT, fill_ones640, 0)
    pltpu.sync_copy(degv, degsh.at[pl.ds(nbase, TPT)])
    plsc.subcore_barrier()

    def scat(j, _):
        pltpu.sync_copy(onesb, degsh.at[colv.at[j]], add=True)
        return 0
    lax.fori_loop(0, CH, scat, 0)
    plsc.subcore_barrier()

    pltpu.sync_copy(degsh.at[pl.ds(nbase, TPT)], degv)

    def newton(r, _):
        d = degv[r, :]
        i = lax.bitcast_convert_type(d, _i32)
        i = jnp.int32(0x5F3759DF) - lax.shift_right_logical(i, 1)
        y = lax.bitcast_convert_type(i, _f32)
        for _ in range(4):
            y = y * (1.5 - 0.5 * d * y * y)
        disv[r, :] = y
        dinvv[r, :] = y * y
        return 0
    lax.fori_loop(0, TPT, newton, 0)

    pltpu.sync_copy(dinvv, dinv2.at[cid, pl.ds(nbase, TPT)])
    pltpu.sync_copy(disv, dis2.at[cid, pl.ds(nbase, TPT)])


_deg_kernel = pl.kernel(
    _deg_body,
    out_type=[jax.ShapeDtypeStruct((2, NP, 16), _f32),
              jax.ShapeDtypeStruct((2, NP, 16), _f32)],
    mesh=_mesh,
    scratch_types=[
        pltpu.VMEM_SHARED((NP, 16), _f32),
        pltpu.VMEM((CH, EW), _i32),
        pltpu.VMEM((EW, 16), _f32),
        pltpu.VMEM((TPT, 16), _f32),
        pltpu.VMEM((TPT, 16), _f32),
        pltpu.VMEM((TPT, 16), _f32),
    ],
    compiler_params=pltpu.CompilerParams(use_tc_tiling_on_sc=False),
)


def _diff_body(xsh, rows_t, cols_t, dinv2, scv_h, outh,
               v_a, v_b, rowg, colg, dinb, dinb2,
               gbufa, gbufb, gbufc, sem_i, sem_g, sem_s, sem_n):
    vv, av = gbufa, gbufb
    cid = lax.axis_index("c")
    sid = lax.axis_index("s")
    nbase = sid * TPT

    pltpu.sync_copy(scv_h, dinb.at[pl.ds(0, K_HOPS + 1)])
    svec = [dinb[k, :] for k in range(K_HOPS + 1)]

    pltpu.sync_copy(xsh.at[cid, pl.ds(nbase, TPT)], v_a.at[pl.ds(nbase, TPT)])
    pltpu.sync_copy(xsh.at[cid, pl.ds(nbase, TPT)], v_b.at[pl.ds(nbase, TPT)])
    plsc.subcore_barrier()

    for k in range(1, K_HOPS + 1):
        src, dst = (v_a, v_b) if k % 2 == 1 else (v_b, v_a)

        ibase = sid * CH
        pltpu.async_copy(rows_t.at[pl.ds(ibase, G)], rowg.at[0], sem_i)
        pltpu.async_copy(cols_t.at[pl.ds(ibase, G)], colg.at[0], sem_i)

        def egroup(g, _, src=src, dst=dst):
            p = lax.rem(g, 2)
            pltpu.make_async_copy(
                rows_t.at[pl.ds(ibase, G)], rowg.at[p], sem_i).wait()
            pltpu.make_async_copy(
                cols_t.at[pl.ds(ibase, G)], colg.at[p], sem_i).wait()

            @pl.when(g < NG - 1)
            def _():
                off = ibase + (g + 1) * G
                pltpu.async_copy(rows_t.at[pl.ds(off, G)],
                                 rowg.at[1 - p], sem_i)
                pltpu.async_copy(cols_t.at[pl.ds(off, G)],
                                 colg.at[1 - p], sem_i)

            gb = (gbufa, gbufb, gbufc)
            gd = [None] * G
            sd = [None] * G
            gd[0] = pltpu.async_copy(src.at[rowg.at[p, 0]], gb[0], sem_g)
            gd[1] = pltpu.async_copy(src.at[rowg.at[p, 1]], gb[1], sem_g)
            for c in range(G):
                gd[c].wait()
                sd[c] = pltpu.async_copy(
                    gb[c % 3], dst.at[colg.at[p, c]], sem_s, add=True)
                if c + 2 < G:
                    if c >= 1:
                        sd[c - 1].wait()
                    gd[c + 2] = pltpu.async_copy(
                        src.at[rowg.at[p, c + 2]], gb[(c + 2) % 3], sem_g)
            for c in range(max(0, G - 3), G):
                sd[c].wait()
            return 0
        lax.fori_loop(0, NG, egroup, 0)
        plsc.subcore_barrier()

        last = (k == K_HOPS)
        avb = (av, gbufc)
        dnb = (dinb, dinb2)
        adesc = [None] * NSUB
        ddesc = [None] * NSUB
        if k > 1:
            adesc[0] = pltpu.async_copy(
                outh.at[cid, pl.ds(nbase, 128)], avb[0].at[pl.ds(0, 128)],
                sem_n)
        if not last:
            ddesc[0] = pltpu.async_copy(
                dinv2.at[cid, pl.ds(nbase, 128)], dnb[0], sem_n)
        for sub in range(NSUB):
            base = nbase + sub * 128
            pb = sub % 2
            pltpu.sync_copy(dst.at[pl.ds(base, 128)], vv.at[pl.ds(0, 128)])
            if sub + 1 < NSUB:
                nxt = nbase + (sub + 1) * 128
                if k > 1:
                    adesc[sub + 1] = pltpu.async_copy(
                        outh.at[cid, pl.ds(nxt, 128)],
                        avb[1 - pb].at[pl.ds(0, 128)], sem_n)
                if not last:
                    ddesc[sub + 1] = pltpu.async_copy(
                        dinv2.at[cid, pl.ds(nxt, 128)], dnb[1 - pb], sem_n)
            if k > 1:
                adesc[sub].wait()
            if not last:
                ddesc[sub].wait()

            def node(r, _, k=k, last=last, ab=avb[pb], db=dnb[pb]):
                dr = db[r, :]
                s = svec[k]
                for fb in range(4):
                    fs = pl.ds(fb * 16, 16)
                    v16 = vv[r, fs]
                    if k == 1:
                        ab[r, fs] = s * v16
                    else:
                        ab[r, fs] = ab[r, fs] + s * v16
                    if not last:
                        vv[r, fs] = dr * v16
                return 0
            lax.fori_loop(0, 128, node, 0)
            pltpu.sync_copy(avb[pb].at[pl.ds(0, 128)],
                            outh.at[cid, pl.ds(base, 128)])
            if not last:
                pltpu.sync_copy(vv.at[pl.ds(0, 128)], dst.at[pl.ds(base, 128)])
                pltpu.sync_copy(vv.at[pl.ds(0, 128)], src.at[pl.ds(base, 128)])
        plsc.subcore_barrier()


_diff_kernel = pl.kernel(
    _diff_body,
    out_type=jax.ShapeDtypeStruct((2, NP, F), _f32),
    mesh=_mesh,
    scratch_types=[
        pltpu.VMEM_SHARED((NP, F), _f32),
        pltpu.VMEM_SHARED((NP, F), _f32),
        pltpu.VMEM((2, G, EW), _i32),
        pltpu.VMEM((2, G, EW), _i32),
        pltpu.VMEM((128, 16), _f32),
        pltpu.VMEM((128, 16), _f32),
        pltpu.VMEM((EW, F), _f32),
        pltpu.VMEM((EW, F), _f32),
        pltpu.VMEM((EW, F), _f32),
        pltpu.SemaphoreType.DMA,
        pltpu.SemaphoreType.DMA,
        pltpu.SemaphoreType.DMA,
        pltpu.SemaphoreType.DMA,
    ],
    compiler_params=pltpu.CompilerParams(use_tc_tiling_on_sc=False),
)


def _mm_body(acc_ref, x_ref, dis_ref, s0_ref, w_ref, b_ref, o_ref, os_ref,
             *, elu, also_scale):
    acc = jnp.concatenate([acc_ref[0], acc_ref[1]], axis=1)
    d = s0_ref[0, 0] * x_ref[...] + dis_ref[...] * acc
    y = lax.dot_general(d, w_ref[...], (((1,), (1,)), ((), ())),
                        preferred_element_type=_f32)
    y = y + b_ref[...]
    if elu:
        y = jnp.where(y > 0, y, jnp.exp(y) - 1.0)
    o_ref[...] = y
    if also_scale:
        ys = dis_ref[...] * y
        os_ref[0] = ys[:, :F]
        os_ref[1] = ys[:, F:]


def _mm(acch, xin, dis, s0, w, b, elu, also_scale):
    blk = 512
    outs = [jax.ShapeDtypeStruct((NP, D), _f32),
            jax.ShapeDtypeStruct((2, NP, F), _f32)]
    o, osh = pl.pallas_call(
        functools.partial(_mm_body, elu=elu, also_scale=also_scale),
        grid=(NP // blk,),
        in_specs=[
            pl.BlockSpec((2, blk, F), lambda i: (0, i, 0)),
            pl.BlockSpec((blk, D), lambda i: (i, 0)),
            pl.BlockSpec((blk, 1), lambda i: (i, 0)),
            pl.BlockSpec((1, 1), lambda i: (0, 0)),
            pl.BlockSpec((D, D), lambda i: (0, 0)),
            pl.BlockSpec((1, D), lambda i: (0, 0)),
        ],
        out_specs=[pl.BlockSpec((blk, D), lambda i: (i, 0)),
                   pl.BlockSpec((2, blk, F), lambda i: (0, i, 0))],
        out_shape=outs,
    )(acch, xin, dis, s0.reshape(1, 1), w, b.reshape(1, D))
    return o, osh


def _prescale_body(x_ref, dis_ref, o_ref):
    ys = dis_ref[...] * x_ref[...]
    o_ref[0] = ys[:, :F]
    o_ref[1] = ys[:, F:]


def _prescale(x, dis):
    blk = 512
    return pl.pallas_call(
        _prescale_body,
        grid=(NP // blk,),
        in_specs=[pl.BlockSpec((blk, D), lambda i: (i, 0)),
                  pl.BlockSpec((blk, 1), lambda i: (i, 0))],
        out_specs=pl.BlockSpec((2, blk, F), lambda i: (0, i, 0)),
        out_shape=jax.ShapeDtypeStruct((2, NP, F), _f32),
    )(x, dis)


def _scales(log_t):
    t = jnp.exp(log_t)
    sc = [jnp.exp(-t)]
    for k in range(1, K_HOPS + 1):
        sc.append(sc[-1] * t / k)
    return jnp.stack(sc)


@jax.jit
def kernel(x, edge_index, log_t1, W1, b1, log_t2, W2, b2):
    ei = jnp.asarray(edge_index, _i32)
    pad = E_PAD - N_EDGES
    fill = jnp.full((pad,), NP - 1, _i32)
    rows_t = jnp.concatenate([ei[0], fill]).reshape(16 * CH, EW)
    cols_t = jnp.concatenate([ei[1], fill]).reshape(16 * CH, EW)

    xp = jnp.pad(x, ((0, NP - N_NODES), (0, 0)))

    dinv2, dis2 = _deg_kernel(cols_t)
    dis_col = dis2[0, :, 0:1]

    s1 = _scales(log_t1)
    s2 = _scales(log_t2)
    sv1 = jnp.broadcast_to(s1[:, None], (K_HOPS + 1, 16))
    sv2 = jnp.broadcast_to(s2[:, None], (K_HOPS + 1, 16))

    xsh1 = _prescale(xp, dis_col)
    acch1 = _diff_kernel(xsh1, rows_t, cols_t, dinv2, sv1)
    h1, xsh2 = _mm(acch1, xp, dis_col, s1[0], W1, b1,
                   elu=True, also_scale=True)
    acch2 = _diff_kernel(xsh2, rows_t, cols_t, dinv2, sv2)
    out, _ = _mm(acch2, h1, dis_col, s2[0], W2, b2,
                 elu=False, also_scale=False)
    return out[:N_NODES]
